# baseline (speedup 1.0000x reference)
import functools

import jax
import jax.numpy as jnp
from jax.experimental import pallas as pl
from jax.experimental.pallas import tpu as pltpu
from jax.experimental.pallas import tpu_sc as plsc

_NBUF = 3
_CHUNK = 2048


def _cos_rows(xb, w, b, ec):
    proj = jax.lax.dot_general(
        xb, w, (((1,), (1,)), ((), ())),
        precision=jax.lax.Precision.DEFAULT,
        preferred_element_type=jnp.float32,
    )
    proj = proj + b
    n = jnp.sqrt(jnp.sum(proj * proj, axis=1, keepdims=True))
    projn = proj / jnp.maximum(n, 1e-12)
    cos = jax.lax.dot_general(
        projn, ec, (((1,), (1,)), ((), ())),
        precision=jax.lax.Precision.DEFAULT,
        preferred_element_type=jnp.float32,
    )
    return cos.T


def _tc_kernel(x_hbm, ec_ref, w_ref, b_ref, cosT_hbm, xbuf, cbuf, sems, osems):
    M = x_hbm.shape[0]
    nchunk = M // _CHUNK

    w = w_ref[...]
    b = b_ref[...]
    ec = ec_ref[...]
    ecn = jnp.sqrt(jnp.sum(ec * ec, axis=1, keepdims=True))
    ecn = ec / jnp.maximum(ecn, 1e-12)

    def copy_in(chunk, slot):
        pltpu.make_async_copy(
            x_hbm.at[pl.ds(chunk * _CHUNK, _CHUNK), :],
            xbuf.at[slot],
            sems.at[slot],
        ).start()

    for j in range(min(_NBUF, nchunk)):
        copy_in(j, j)

    def body(i, carry):
        slot = jax.lax.rem(i, _NBUF)
        oslot = jax.lax.rem(i, 2)
        pltpu.make_async_copy(
            x_hbm.at[pl.ds(i * _CHUNK, _CHUNK), :],
            xbuf.at[slot],
            sems.at[slot],
        ).wait()
        cosT = _cos_rows(xbuf[slot], w, b, ecn)

        @pl.when(i >= 2)
        def _():
            pltpu.make_async_copy(
                cbuf.at[oslot], cosT_hbm.at[:, pl.ds(0, _CHUNK)],
                osems.at[oslot],
            ).wait()

        cbuf[oslot] = cosT
        pltpu.make_async_copy(
            cbuf.at[oslot],
            cosT_hbm.at[:, pl.ds(i * _CHUNK, _CHUNK)],
            osems.at[oslot],
        ).start()

        @pl.when(i + _NBUF < nchunk)
        def _():
            copy_in(i + _NBUF, slot)

        return carry

    jax.lax.fori_loop(0, nchunk, body, 0)
    for t in range(2):
        pltpu.make_async_copy(
            cbuf.at[t], cosT_hbm.at[:, pl.ds(0, _CHUNK)], osems.at[t],
        ).wait()


def _tc_cosine(x2, expert_centers, W, b2):
    M, T2 = x2.shape
    C, E = expert_centers.shape
    return pl.pallas_call(
        _tc_kernel,
        in_specs=[
            pl.BlockSpec(memory_space=pl.ANY),
            pl.BlockSpec(memory_space=pltpu.VMEM),
            pl.BlockSpec(memory_space=pltpu.VMEM),
            pl.BlockSpec(memory_space=pltpu.VMEM),
        ],
        out_specs=pl.BlockSpec(memory_space=pl.ANY),
        out_shape=jax.ShapeDtypeStruct((C, M), jnp.float32),
        scratch_shapes=[
            pltpu.VMEM((_NBUF, _CHUNK, T2), jnp.float32),
            pltpu.VMEM((2, C, _CHUNK), jnp.float32),
            pltpu.SemaphoreType.DMA((_NBUF,)),
            pltpu.SemaphoreType.DMA((2,)),
        ],
    )(x2, expert_centers, W, b2)


_SC_LANES = 16
_SC_BT = 512


def _sc_router(cosT):
    C, M = cosT.shape
    mesh = plsc.VectorSubcoreMesh(core_axis_name="core",
                                  subcore_axis_name="subcore")

    @pl.kernel(out_type=jax.ShapeDtypeStruct((4, M), jnp.float32), mesh=mesh)
    def sc_kernel(cos_hbm, out_hbm):
        def body(cos_vmem, out_vmem):
            @pl.loop(0, _SC_BT, step=_SC_LANES)
            def _(c):
                sl = pl.ds(c, _SC_LANES)
                m1 = cos_vmem[0:1, sl]
                i1 = jnp.zeros((1, _SC_LANES), jnp.float32)
                m2 = jnp.full((1, _SC_LANES), -jnp.inf, jnp.float32)
                i2 = jnp.zeros((1, _SC_LANES), jnp.float32)

                def step(e, carry):
                    m1, i1, m2, i2 = carry
                    v = cos_vmem[pl.ds(e, 1), sl]
                    ef = jnp.full((1, _SC_LANES), 1.0, jnp.float32) * e.astype(jnp.float32)
                    gt1 = v > m1
                    gt2 = v > m2
                    m2n = jnp.where(gt1, m1, jnp.where(gt2, v, m2))
                    i2n = jnp.where(gt1, i1, jnp.where(gt2, ef, i2))
                    m1n = jnp.where(gt1, v, m1)
                    i1n = jnp.where(gt1, ef, i1)
                    return m1n, i1n, m2n, i2n

                m1, i1, m2, i2 = jax.lax.fori_loop(
                    1, C, step, (m1, i1, m2, i2))
                e_ = jnp.exp(m2 - m1)
                den = 1.0 + e_
                out_vmem[0:1, sl] = 1.0 / den
                out_vmem[1:2, sl] = e_ / den
                out_vmem[2:3, sl] = i1
                out_vmem[3:4, sl] = i2

        pltpu.emit_pipeline(
            body,
            grid=(M // _SC_BT,),
            in_specs=[pl.BlockSpec((C, _SC_BT), lambda i: (0, i))],
            out_specs=[pl.BlockSpec((4, _SC_BT), lambda i: (0, i))],
            core_axis_name=("core", "subcore"),
            dimension_semantics=(pltpu.PARALLEL,),
        )(cos_hbm, out_hbm)

    return sc_kernel(cosT)


@functools.partial(jax.jit, static_argnames=())
def kernel(x, expert_centers, W, b):
    bs, C, T2 = x.shape
    E = W.shape[0]
    M = bs * C
    x2 = x.reshape(M, T2)
    b2 = b.reshape(1, E)
    cosT = _tc_cosine(x2, expert_centers, W, b2)
    packed = _sc_router(cosT)
    probs = packed[0:2].T.reshape(bs, C, 2)
    idx = packed[2:4].astype(jnp.int32).T.reshape(bs, C, 2)
    return probs, idx


# speedup vs baseline: 1.3245x; 1.3245x over previous
import functools

import jax
import jax.numpy as jnp
from jax.experimental import pallas as pl
from jax.experimental.pallas import tpu as pltpu

_NBUF = 3
_CHUNK = 2048


def _route_rows(xb, w, b, ec):
    proj = jax.lax.dot_general(
        xb, w, (((1,), (1,)), ((), ())),
        precision=jax.lax.Precision.DEFAULT,
        preferred_element_type=jnp.float32,
    )
    proj = proj + b
    n = jnp.sqrt(jnp.sum(proj * proj, axis=1, keepdims=True))
    projn = proj / jnp.maximum(n, 1e-12)

    cos = jax.lax.dot_general(
        projn, ec, (((1,), (1,)), ((), ())),
        precision=jax.lax.Precision.DEFAULT,
        preferred_element_type=jnp.float32,
    )

    C = cos.shape[1]
    iota = jax.lax.broadcasted_iota(jnp.int32, cos.shape, 1).astype(jnp.float32)
    m1 = jnp.max(cos, axis=1, keepdims=True)
    i1 = jnp.min(jnp.where(cos == m1, iota, float(C)), axis=1, keepdims=True)
    cos2 = jnp.where(iota == i1, -jnp.inf, cos)
    m2 = jnp.max(cos2, axis=1, keepdims=True)
    i2 = jnp.min(jnp.where(cos2 == m2, iota, float(C)), axis=1, keepdims=True)

    e = jnp.exp(m2 - m1)
    denom = 1.0 + e
    p1 = 1.0 / denom
    p2 = e / denom
    packed = jnp.concatenate([p1, p2, i1, i2], axis=1).T
    probs = packed[0:2]
    idx = packed[2:4].astype(jnp.int32)
    return probs, idx


def _router_kernel(x_hbm, ec_ref, w_ref, b_ref, probs_ref, idx_ref,
                   xbuf, sems):
    M = x_hbm.shape[0]
    nchunk = M // _CHUNK

    w = w_ref[...]
    b = b_ref[...]
    ec = ec_ref[...]
    ecn = jnp.sqrt(jnp.sum(ec * ec, axis=1, keepdims=True))
    ecn = ec / jnp.maximum(ecn, 1e-12)

    half = _CHUNK // 2

    def copy_in(chunk, slot):
        pltpu.make_async_copy(
            x_hbm.at[pl.ds(chunk * _CHUNK, half), :],
            xbuf.at[slot, pl.ds(0, half), :],
            sems.at[slot, 0],
        ).start()
        pltpu.make_async_copy(
            x_hbm.at[pl.ds(chunk * _CHUNK + half, half), :],
            xbuf.at[slot, pl.ds(half, half), :],
            sems.at[slot, 1],
        ).start()

    for j in range(min(_NBUF, nchunk)):
        copy_in(j, j)

    def body(i, carry):
        slot = jax.lax.rem(i, _NBUF)
        pltpu.make_async_copy(
            x_hbm.at[pl.ds(i * _CHUNK, half), :],
            xbuf.at[slot, pl.ds(0, half), :],
            sems.at[slot, 0],
        ).wait()
        pltpu.make_async_copy(
            x_hbm.at[pl.ds(i * _CHUNK + half, half), :],
            xbuf.at[slot, pl.ds(half, half), :],
            sems.at[slot, 1],
        ).wait()
        p, ix = _route_rows(xbuf[slot], w, b, ecn)
        probs_ref[:, pl.ds(i * _CHUNK, _CHUNK)] = p
        idx_ref[:, pl.ds(i * _CHUNK, _CHUNK)] = ix

        @pl.when(i + _NBUF < nchunk)
        def _():
            copy_in(i + _NBUF, slot)

        return carry

    jax.lax.fori_loop(0, nchunk, body, 0)


@functools.partial(jax.jit, static_argnames=())
def kernel(x, expert_centers, W, b):
    bs, C, T2 = x.shape
    E = W.shape[0]
    M = bs * C
    x2 = x.reshape(M, T2)
    b2 = b.reshape(1, E)
    probs2, idx2 = pl.pallas_call(
        _router_kernel,
        in_specs=[
            pl.BlockSpec(memory_space=pl.ANY),
            pl.BlockSpec(memory_space=pltpu.VMEM),
            pl.BlockSpec(memory_space=pltpu.VMEM),
            pl.BlockSpec(memory_space=pltpu.VMEM),
        ],
        out_specs=[
            pl.BlockSpec(memory_space=pltpu.VMEM),
            pl.BlockSpec(memory_space=pltpu.VMEM),
        ],
        out_shape=[
            jax.ShapeDtypeStruct((2, M), jnp.float32),
            jax.ShapeDtypeStruct((2, M), jnp.int32),
        ],
        scratch_shapes=[
            pltpu.VMEM((_NBUF, _CHUNK, T2), jnp.float32),
            pltpu.SemaphoreType.DMA((_NBUF, 2)),
        ],
    )(x2, expert_centers, W, b2)
    return (probs2.T.reshape(bs, C, 2), idx2.T.reshape(bs, C, 2))
